# baseline (speedup 1.0000x reference)
import functools

import jax
import jax.numpy as jnp
from jax import lax
from jax.experimental import pallas as pl
from jax.experimental.pallas import tpu as pltpu
from jax.experimental.pallas import tpu_sc as plsc

NC = 2
NS = 16
LANES = 16
NW = NC * NS
NPIECE = 8


def _align8(n):
    return ((n + 7) // 8) * 8


def _table_layout(E):
    t3 = E * E * E
    poff = _align8(t3)
    soff = _align8(poff + E * E)
    stride = _align8(soff + E)
    return poff, soff, stride


@functools.partial(jax.jit, static_argnums=(2, 3, 4, 5))
def _embedding_bag_mean(x_words, tbl_flat, B, L, E, D):
    chunk = B // NW
    nwords = L // 4
    nblk = L // 12
    POFF, SOFF, STRIDE = _table_layout(E)

    rem = list(range(12 * nblk, L))
    rem_trips = [tuple(rem[i: i + 3]) for i in range(0, len(rem) - 2, 3)]
    rem2 = rem[3 * len(rem_trips):]
    rem_pairs = [tuple(rem2[i: i + 2]) for i in range(0, len(rem2) - 1, 2)]
    rem_sing = rem2[2 * len(rem_pairs):]

    mesh = plsc.VectorSubcoreMesh(core_axis_name="c", subcore_axis_name="s")

    @functools.partial(
        pl.kernel,
        out_type=jax.ShapeDtypeStruct((B * D,), jnp.float32),
        mesh=mesh,
        scratch_types=[
            pltpu.VMEM((chunk * nwords,), jnp.int32),
            pltpu.VMEM((chunk * D,), jnp.float32),
            pltpu.VMEM((D * STRIDE,), jnp.float32),
            pltpu.SemaphoreType.DMA,
            pltpu.SemaphoreType.DMA,
        ],
        compiler_params=pltpu.CompilerParams(needs_layout_passes=False),
    )
    def sc_kernel(x_hbm, tbl_hbm, out_hbm, x_v, out_v, tbl_v, sem_t, sem_x):
        wid = lax.axis_index("s") * NC + lax.axis_index("c")
        base = wid * chunk
        wbase = base * nwords
        pw = chunk * nwords // NPIECE

        pltpu.sync_copy(tbl_hbm, tbl_v)
        pltpu.sync_copy(x_hbm.at[pl.ds(wbase, chunk * nwords)], x_v)

        tsub = [tbl_v.at[pl.ds(d * STRIDE, E * E * E)] for d in range(D)]
        psub = [tbl_v.at[pl.ds(d * STRIDE + POFF, E * E)] for d in range(D)]
        ssub = [tbl_v.at[pl.ds(d * STRIDE + SOFF, E)] for d in range(D)]

        lane = lax.iota(jnp.int32, LANES)
        scale = jnp.float32(1.0 / L)
        e_vec = jnp.full((LANES,), E, jnp.int32)
        gpp = chunk // NPIECE // LANES

        def peel(word, p):
            sh = 8 * (p % 4)
            return word >> 24 if sh == 24 else (word >> sh) & 255

        def group_body(g, _):
            rows = g * LANES + lane
            rows_w = rows * nwords

            def blk_body(k, accs):
                accs = list(accs)
                wb = 3 * k
                w0 = plsc.load_gather(x_v, [rows_w + wb])
                w1 = plsc.load_gather(x_v, [rows_w + (wb + 1)])
                w2 = plsc.load_gather(x_v, [rows_w + (wb + 2)])
                byts = [peel(w0, p) for p in range(4)]
                byts += [peel(w1, p) for p in range(4)]
                byts += [peel(w2, p) for p in range(4)]
                for t in range(4):
                    tidx = (byts[3 * t] * e_vec + byts[3 * t + 1]) * e_vec \
                        + byts[3 * t + 2]
                    for d in range(D):
                        accs[d] = accs[d] + plsc.load_gather(tsub[d], [tidx])
                return tuple(accs)

            accs = lax.fori_loop(
                0, nblk, blk_body,
                tuple(jnp.zeros((LANES,), jnp.float32) for _ in range(D)),
            )
            accs = list(accs)

            if rem:
                wv = {
                    wd: plsc.load_gather(x_v, [rows_w + wd])
                    for wd in sorted({p // 4 for p in rem})
                }
                for (a, b, c) in rem_trips:
                    tidx = (peel(wv[a // 4], a) * e_vec
                            + peel(wv[b // 4], b)) * e_vec \
                        + peel(wv[c // 4], c)
                    for d in range(D):
                        accs[d] = accs[d] + plsc.load_gather(tsub[d], [tidx])
                for (a, b) in rem_pairs:
                    pidx = peel(wv[a // 4], a) * e_vec + peel(wv[b // 4], b)
                    for d in range(D):
                        accs[d] = accs[d] + plsc.load_gather(psub[d], [pidx])
                for a in rem_sing:
                    sidx = peel(wv[a // 4], a)
                    for d in range(D):
                        accs[d] = accs[d] + plsc.load_gather(ssub[d], [sidx])

            out_base = rows * D
            for d in range(D):
                plsc.store_scatter(out_v, [out_base + d], accs[d] * scale)
            return 0

        lax.fori_loop(0, NPIECE * gpp, group_body, 0)

        pltpu.sync_copy(out_v, out_hbm.at[pl.ds(base * D, chunk * D)])

    return sc_kernel(x_words, tbl_flat)


def kernel(x, weight):
    B, L = x.shape
    E, D = weight.shape
    x_words = lax.bitcast_convert_type(
        x.astype(jnp.int8).reshape(B * L // 4, 4), jnp.int32
    )
    w = weight.astype(jnp.float32)
    poff, soff, stride = _table_layout(E)
    pairs = (w[:, None, :] + w[None, :, :]).reshape(E * E, D)
    trips = (pairs[:, None, :] + w[None, :, :]).reshape(E * E * E, D)
    tbl = (
        jnp.zeros((D, stride), jnp.float32)
        .at[:, : E * E * E].set(trips.T)
        .at[:, poff: poff + E * E].set(pairs.T)
        .at[:, soff: soff + E].set(w.T)
        .reshape(-1)
    )
    out = _embedding_bag_mean(x_words, tbl, B, L, E, D)
    return out.reshape(B, D)


# speedup vs baseline: 4.5184x; 4.5184x over previous
import functools

import jax
import jax.numpy as jnp
from jax import lax
from jax.experimental import pallas as pl
from jax.experimental.pallas import tpu as pltpu
from jax.experimental.pallas import tpu_sc as plsc

NC = 2
NS = 16
LANES = 16
NW = NC * NS
NPIECE = 8


def _align8(n):
    return ((n + 7) // 8) * 8


def _table_layout(E):
    t3 = E * E * E
    poff = _align8(t3)
    soff = _align8(poff + E * E)
    stride = _align8(soff + E)
    return poff, soff, stride


@functools.partial(jax.jit, static_argnums=(2, 3, 4, 5))
def _embedding_bag_mean(x_words, tbl_flat, B, L, E, D):
    chunk = B // NW
    nwords = L // 4
    nblk = nwords // 3
    POFF, SOFF, STRIDE = _table_layout(E)

    nrem_w = nwords % 3
    rem_bytes = [(3 * nblk + w, j) for w in range(nrem_w) for j in range(4)]
    rem_trips = [tuple(rem_bytes[i: i + 3])
                 for i in range(0, len(rem_bytes) - 2, 3)]
    rem2 = rem_bytes[3 * len(rem_trips):]
    rem_pairs = [tuple(rem2[i: i + 2]) for i in range(0, len(rem2) - 1, 2)]
    rem_sing = rem2[2 * len(rem_pairs):]

    mesh = plsc.VectorSubcoreMesh(core_axis_name="c", subcore_axis_name="s")

    @functools.partial(
        pl.kernel,
        out_type=jax.ShapeDtypeStruct((B * D,), jnp.float32),
        mesh=mesh,
        scratch_types=[
            pltpu.VMEM((chunk * nwords,), jnp.int32),
            pltpu.VMEM((chunk * D,), jnp.float32),
            pltpu.VMEM((D * STRIDE,), jnp.float32),
            pltpu.SemaphoreType.DMA,
            pltpu.SemaphoreType.DMA,
        ],
        compiler_params=pltpu.CompilerParams(needs_layout_passes=False),
    )
    def sc_kernel(x_hbm, tbl_hbm, out_hbm, x_v, out_v, tbl_v, sem_t, sem_x):
        wid = lax.axis_index("s") * NC + lax.axis_index("c")
        base = wid * chunk
        wbase = base * nwords
        pw = chunk * nwords // NPIECE

        tcopy = pltpu.make_async_copy(tbl_hbm, tbl_v, sem_t)
        tcopy.start()
        pieces = [
            pltpu.make_async_copy(
                x_hbm.at[pl.ds(wbase + r * pw, pw)],
                x_v.at[pl.ds(r * pw, pw)],
                sem_x,
            )
            for r in range(NPIECE)
        ]
        for c in pieces:
            c.start()
        tcopy.wait()

        tsub = [tbl_v.at[pl.ds(d * STRIDE, E * E * E)] for d in range(D)]
        psub = [tbl_v.at[pl.ds(d * STRIDE + POFF, E * E)] for d in range(D)]
        ssub = [tbl_v.at[pl.ds(d * STRIDE + SOFF, E)] for d in range(D)]

        lane = lax.iota(jnp.int32, LANES)
        scale = jnp.float32(1.0 / L)
        e_vec = jnp.full((LANES,), E, jnp.int32)
        gpp = chunk // NPIECE // LANES

        def peel(word, j):
            sh = 8 * j
            return word >> 24 if sh == 24 else (word >> sh) & 255

        def group_body(g, _):
            rows = g * LANES + lane
            rows_w = rows * nwords

            def blk_body(k, accs):
                accs = list(accs)
                wb = 3 * k
                w0 = plsc.load_gather(x_v, [rows_w + wb])
                w1 = plsc.load_gather(x_v, [rows_w + (wb + 1)])
                w2 = plsc.load_gather(x_v, [rows_w + (wb + 2)])
                byts = [peel(w, j) for w in (w0, w1, w2) for j in range(4)]
                for t in range(4):
                    tidx = (byts[3 * t] * e_vec + byts[3 * t + 1]) * e_vec \
                        + byts[3 * t + 2]
                    for d in range(D):
                        accs[d] = accs[d] + plsc.load_gather(tsub[d], [tidx])
                return tuple(accs)

            accs = lax.fori_loop(
                0, nblk, blk_body,
                tuple(jnp.zeros((LANES,), jnp.float32) for _ in range(D)),
            )
            accs = list(accs)

            if rem_bytes:
                wv = {
                    wd: plsc.load_gather(x_v, [rows_w + wd])
                    for wd in sorted({w for (w, _) in rem_bytes})
                }
                for (a, b, c) in rem_trips:
                    tidx = (peel(wv[a[0]], a[1]) * e_vec
                            + peel(wv[b[0]], b[1])) * e_vec \
                        + peel(wv[c[0]], c[1])
                    for d in range(D):
                        accs[d] = accs[d] + plsc.load_gather(tsub[d], [tidx])
                for (a, b) in rem_pairs:
                    pidx = peel(wv[a[0]], a[1]) * e_vec + peel(wv[b[0]], b[1])
                    for d in range(D):
                        accs[d] = accs[d] + plsc.load_gather(psub[d], [pidx])
                for a in rem_sing:
                    sidx = peel(wv[a[0]], a[1])
                    for d in range(D):
                        accs[d] = accs[d] + plsc.load_gather(ssub[d], [sidx])

            out_base = rows * D
            for d in range(D):
                plsc.store_scatter(out_v, [out_base + d], accs[d] * scale)
            return 0

        for r in range(NPIECE):
            pieces[r].wait()
            lax.fori_loop(r * gpp, (r + 1) * gpp, group_body, 0)

        pltpu.sync_copy(out_v, out_hbm.at[pl.ds(base * D, chunk * D)])

    return sc_kernel(x_words, tbl_flat)


def kernel(x, weight):
    B, L = x.shape
    E, D = weight.shape
    xr = x.astype(jnp.int32).reshape(B, 4, L // 4)
    x_words = (
        xr[:, 0, :] + xr[:, 1, :] * 256
        + xr[:, 2, :] * 65536 + xr[:, 3, :] * 16777216
    ).reshape(B * (L // 4))
    w = weight.astype(jnp.float32)
    poff, soff, stride = _table_layout(E)
    pairs = (w[:, None, :] + w[None, :, :]).reshape(E * E, D)
    trips = (pairs[:, None, :] + w[None, :, :]).reshape(E * E * E, D)
    tbl = (
        jnp.zeros((D, stride), jnp.float32)
        .at[:, : E * E * E].set(trips.T)
        .at[:, poff: poff + E * E].set(pairs.T)
        .at[:, soff: soff + E].set(w.T)
        .reshape(-1)
    )
    out = _embedding_bag_mean(x_words, tbl, B, L, E, D)
    return out.reshape(B, D)
